# baseline (speedup 1.0000x reference)
import functools

import jax
import jax.numpy as jnp
from jax import lax
from jax.experimental import pallas as pl
from jax.experimental.pallas import tpu as pltpu
from jax.experimental.pallas import tpu_sc as plsc

_B = 16384
_DIM = 64
_NW = 32
_BPW = _B // _NW
_CH = 128
_NCH = _BPW // _CH

def _q2b_sc_body(ent_hbm, attr_hbm, ent_emb, attr_emb, off_emb,
                 out_hbm, eidx, aidx, e_v, a_v, o_v, out_v, sem):
    wid = lax.axis_index("s") * 2 + lax.axis_index("c")
    base = wid * _BPW

    for j in range(_NCH):
        pltpu.sync_copy(ent_hbm.at[pl.ds(base + j * _CH, _CH)], eidx.at[j])
        pltpu.sync_copy(attr_hbm.at[pl.ds(base + j * _CH, _CH)], aidx.at[j])

    copies = []
    for j in range(_NCH):
        dst = pl.ds(j * _CH, _CH)
        copies.append(pltpu.async_copy(ent_emb.at[eidx.at[j]], e_v.at[dst], sem))
        copies.append(pltpu.async_copy(attr_emb.at[aidx.at[j]], a_v.at[dst], sem))
        copies.append(pltpu.async_copy(off_emb.at[aidx.at[j]], o_v.at[dst], sem))
    for c in copies:
        c.wait()

    lane = lax.iota(jnp.int32, 16)
    last = lane == 15

    def row(i, _):
        acc_d = jnp.zeros((16,), jnp.float32)
        acc_o = jnp.zeros((16,), jnp.float32)
        for c in range(_DIM // 16):
            sl = pl.ds(c * 16, 16)
            e = e_v[i, sl]
            a = a_v[i, sl]
            o = o_v[i, sl]
            acc_d = acc_d + jnp.maximum(o - jnp.abs(e - a), 0.0)
            acc_o = acc_o + jnp.abs(o)
        sd = jnp.cumsum(acc_d)
        so = jnp.cumsum(acc_o)
        res = 1.0 - sd / so
        plsc.store_scatter(out_v, [jnp.full((16,), i, jnp.int32)], res, mask=last)
        return ()

    lax.fori_loop(0, _BPW, row, (), unroll=2)

    pltpu.sync_copy(out_v, out_hbm.at[pl.ds(base, _BPW)])


@functools.cache
def _build():
    mesh = plsc.VectorSubcoreMesh(core_axis_name="c", subcore_axis_name="s")
    return pl.kernel(
        _q2b_sc_body,
        mesh=mesh,
        out_type=jax.ShapeDtypeStruct((_B,), jnp.float32),
        scratch_types=[
            pltpu.VMEM((_NCH, _CH), jnp.int32),
            pltpu.VMEM((_NCH, _CH), jnp.int32),
            pltpu.VMEM((_BPW, _DIM), jnp.float32),
            pltpu.VMEM((_BPW, _DIM), jnp.float32),
            pltpu.VMEM((_BPW, _DIM), jnp.float32),
            pltpu.VMEM((_BPW,), jnp.float32),
            pltpu.SemaphoreType.DMA,
        ],
        compiler_params=pltpu.CompilerParams(
            needs_layout_passes=False, use_tc_tiling_on_sc=False),
    )


def kernel(entities, attributes, ent_emb, attr_emb, offset_attr_emb):
    return _build()(entities, attributes, ent_emb, attr_emb, offset_attr_emb)
